# baseline (speedup 1.0000x reference)
import functools

import jax
import jax.numpy as jnp
from jax import lax
from jax.experimental import pallas as pl
from jax.experimental.pallas import tpu as pltpu
from jax.experimental.pallas import tpu_sc as plsc

_NUM_CORES = 2
_NUM_SUBCORES = 16
_NUM_WORKERS = _NUM_CORES * _NUM_SUBCORES
_CHUNK = 128


def _gather_call(B, V, D, t, pos_encoding):
    b_per_w = B // _NUM_WORKERS
    n_chunks = b_per_w // _CHUNK
    mesh = plsc.VectorSubcoreMesh(core_axis_name="c", subcore_axis_name="s")

    @functools.partial(
        pl.kernel,
        mesh=mesh,
        out_type=jax.ShapeDtypeStruct((B, D), jnp.float32),
        scratch_types=[
            pltpu.VMEM((b_per_w,), jnp.int32),
            pltpu.VMEM((b_per_w, D), jnp.float32),
            pltpu.VMEM_SHARED((V, D), jnp.float32),
            pltpu.SemaphoreType.DMA,
            pltpu.SemaphoreType.DMA,
        ],
    )
    def gather_kernel(t_hbm, table_hbm, out_hbm, idx_v, rows_v, table_s,
                      gsem, osem):
        sid = lax.axis_index("s")
        wid = sid * _NUM_CORES + lax.axis_index("c")
        base = wid * b_per_w
        @pl.when(sid == 0)
        def _():
            pltpu.sync_copy(table_hbm, table_s)
        pltpu.sync_copy(t_hbm.at[pl.ds(base, b_per_w)], idx_v)
        plsc.subcore_barrier()
        gathers = [
            pltpu.async_copy(table_s.at[idx_v],
                             rows_v, gsem)
        ]
        gathers[0].wait()
        pltpu.async_copy(rows_v, out_hbm.at[pl.ds(base, b_per_w)], osem).wait()

    return gather_kernel(t, pos_encoding)


def kernel(t, pos_encoding):
    B = t.shape[0]
    V, D = pos_encoding.shape
    t = t.astype(jnp.int32)
    pos_encoding = pos_encoding.astype(jnp.float32)
    return _gather_call(B, V, D, t, pos_encoding)


# speedup vs baseline: 1.0467x; 1.0467x over previous
import functools

import jax
import jax.numpy as jnp
from jax import lax
from jax.experimental import pallas as pl
from jax.experimental.pallas import tpu as pltpu
from jax.experimental.pallas import tpu_sc as plsc

_NUM_CORES = 2
_NUM_SUBCORES = 16
_NUM_WORKERS = _NUM_CORES * _NUM_SUBCORES
_CHUNK = 64


def _gather_call(B, V, D, t, pos_encoding):
    b_per_w = B // _NUM_WORKERS
    n_chunks = b_per_w // _CHUNK
    mesh = plsc.VectorSubcoreMesh(core_axis_name="c", subcore_axis_name="s")

    @functools.partial(
        pl.kernel,
        mesh=mesh,
        out_type=jax.ShapeDtypeStruct((B, D), jnp.float32),
        scratch_types=[
            pltpu.VMEM((b_per_w,), jnp.int32),
            pltpu.VMEM((b_per_w, D), jnp.float32),
            pltpu.VMEM_SHARED((V, D), jnp.float32),
            pltpu.SemaphoreType.DMA,
            pltpu.SemaphoreType.DMA,
        ],
    )
    def gather_kernel(t_hbm, table_hbm, out_hbm, idx_v, rows_v, table_s,
                      gsem, osem):
        sid = lax.axis_index("s")
        wid = sid * _NUM_CORES + lax.axis_index("c")
        base = wid * b_per_w
        @pl.when(sid == 0)
        def _():
            pltpu.sync_copy(table_hbm, table_s)
        pltpu.sync_copy(t_hbm.at[pl.ds(base, b_per_w)], idx_v)
        plsc.subcore_barrier()
        gathers = [
            pltpu.async_copy(table_s.at[idx_v.at[pl.ds(j * _CHUNK, _CHUNK)]],
                             rows_v.at[pl.ds(j * _CHUNK, _CHUNK)], gsem)
            for j in range(n_chunks)
        ]
        outs = []
        for j in range(n_chunks):
            gathers[j].wait()
            outs.append(
                pltpu.async_copy(rows_v.at[pl.ds(j * _CHUNK, _CHUNK)],
                                 out_hbm.at[pl.ds(base + j * _CHUNK, _CHUNK)],
                                 osem))
        for c in outs:
            c.wait()

    return gather_kernel(t, pos_encoding)


def kernel(t, pos_encoding):
    B = t.shape[0]
    V, D = pos_encoding.shape
    t = t.astype(jnp.int32)
    pos_encoding = pos_encoding.astype(jnp.float32)
    return _gather_call(B, V, D, t, pos_encoding)


# speedup vs baseline: 1.0584x; 1.0112x over previous
import functools

import jax
import jax.numpy as jnp
from jax import lax
from jax.experimental import pallas as pl
from jax.experimental.pallas import tpu as pltpu
from jax.experimental.pallas import tpu_sc as plsc

_NUM_CORES = 2
_NUM_SUBCORES = 16
_NUM_WORKERS = _NUM_CORES * _NUM_SUBCORES
_CHUNK = 64


def _gather_call(B, V, D, t, pos_encoding):
    b_per_w = B // _NUM_WORKERS
    n_chunks = b_per_w // _CHUNK
    mesh = plsc.VectorSubcoreMesh(core_axis_name="c", subcore_axis_name="s")

    @functools.partial(
        pl.kernel,
        mesh=mesh,
        out_type=jax.ShapeDtypeStruct((B, D), jnp.float32),
        scratch_types=[
            pltpu.VMEM((b_per_w,), jnp.int32),
            pltpu.VMEM((b_per_w, D), jnp.float32),
            pltpu.VMEM_SHARED((V, D), jnp.float32),
            pltpu.SemaphoreType.DMA,
            pltpu.SemaphoreType.DMA,
        ],
    )
    def gather_kernel(t_hbm, table_hbm, out_hbm, idx_v, rows_v, table_s,
                      gsem, osem):
        sid = lax.axis_index("s")
        wid = sid * _NUM_CORES + lax.axis_index("c")
        base = wid * b_per_w
        rows_per_tile = -(-V // _NUM_SUBCORES)
        rows_per_tile = -(-rows_per_tile // 8) * 8
        toff = jnp.minimum(sid * rows_per_tile, (V - rows_per_tile) // 8 * 8)
        stage = pltpu.async_copy(table_hbm.at[pl.ds(toff, rows_per_tile)],
                                 table_s.at[pl.ds(toff, rows_per_tile)], osem)
        pltpu.sync_copy(t_hbm.at[pl.ds(base, b_per_w)], idx_v)
        stage.wait()
        plsc.subcore_barrier()
        gathers = [
            pltpu.async_copy(table_s.at[idx_v.at[pl.ds(j * _CHUNK, _CHUNK)]],
                             rows_v.at[pl.ds(j * _CHUNK, _CHUNK)], gsem)
            for j in range(n_chunks)
        ]
        outs = []
        for j in range(n_chunks):
            gathers[j].wait()
            outs.append(
                pltpu.async_copy(rows_v.at[pl.ds(j * _CHUNK, _CHUNK)],
                                 out_hbm.at[pl.ds(base + j * _CHUNK, _CHUNK)],
                                 osem))
        for c in outs:
            c.wait()

    return gather_kernel(t, pos_encoding)


def kernel(t, pos_encoding):
    B = t.shape[0]
    V, D = pos_encoding.shape
    t = t.astype(jnp.int32)
    pos_encoding = pos_encoding.astype(jnp.float32)
    return _gather_call(B, V, D, t, pos_encoding)


# speedup vs baseline: 1.0763x; 1.0169x over previous
import functools

import jax
import jax.numpy as jnp
from jax import lax
from jax.experimental import pallas as pl
from jax.experimental.pallas import tpu as pltpu
from jax.experimental.pallas import tpu_sc as plsc

_NUM_CORES = 2
_NUM_SUBCORES = 16
_NUM_WORKERS = _NUM_CORES * _NUM_SUBCORES
_CHUNK = 64


def _gather_call(B, V, D, t, pos_encoding):
    b_per_w = B // _NUM_WORKERS
    n_chunks = b_per_w // _CHUNK
    mesh = plsc.VectorSubcoreMesh(core_axis_name="c", subcore_axis_name="s")

    @functools.partial(
        pl.kernel,
        mesh=mesh,
        out_type=jax.ShapeDtypeStruct((B, D), jnp.float32),
        scratch_types=[
            pltpu.VMEM((b_per_w,), jnp.int32),
            pltpu.VMEM((b_per_w, D), jnp.float32),
            pltpu.VMEM_SHARED((V, D), jnp.float32),
            pltpu.SemaphoreType.DMA,
            pltpu.SemaphoreType.DMA,
        ],
    )
    def gather_kernel(t_hbm, table_hbm, out_hbm, idx_v, rows_v, table_s,
                      gsem, osem):
        sid = lax.axis_index("s")
        wid = sid * _NUM_CORES + lax.axis_index("c")
        base = wid * b_per_w
        rows_per_tile = -(-V // _NUM_SUBCORES)
        rows_per_tile = -(-rows_per_tile // 8) * 8
        toff = jnp.minimum(sid * rows_per_tile, (V - rows_per_tile) // 8 * 8)
        stage = pltpu.async_copy(table_hbm.at[pl.ds(toff, rows_per_tile)],
                                 table_s.at[pl.ds(toff, rows_per_tile)], osem)
        pltpu.sync_copy(t_hbm.at[pl.ds(base, b_per_w)], idx_v)
        gathers = [
            pltpu.async_copy(table_hbm.at[idx_v.at[pl.ds(0, _CHUNK)]],
                             rows_v.at[pl.ds(0, _CHUNK)], gsem)
        ]
        stage.wait()
        plsc.subcore_barrier()
        gathers += [
            pltpu.async_copy(table_s.at[idx_v.at[pl.ds(j * _CHUNK, _CHUNK)]],
                             rows_v.at[pl.ds(j * _CHUNK, _CHUNK)], gsem)
            for j in range(1, n_chunks)
        ]
        outs = []
        for j in range(n_chunks):
            gathers[j].wait()
            outs.append(
                pltpu.async_copy(rows_v.at[pl.ds(j * _CHUNK, _CHUNK)],
                                 out_hbm.at[pl.ds(base + j * _CHUNK, _CHUNK)],
                                 osem))
        for c in outs:
            c.wait()

    return gather_kernel(t, pos_encoding)


def kernel(t, pos_encoding):
    B = t.shape[0]
    V, D = pos_encoding.shape
    t = t.astype(jnp.int32)
    pos_encoding = pos_encoding.astype(jnp.float32)
    return _gather_call(B, V, D, t, pos_encoding)
